# baseline (speedup 1.0000x reference)
import functools

import jax
import jax.numpy as jnp
from jax import lax
from jax.experimental import pallas as pl
from jax.experimental.pallas import tpu as pltpu
from jax.experimental.pallas import tpu_sc as plsc

_C = 128
_NBUF = 2
_D = 128
_V = 64


def _sc_body(x1_hbm, x2_hbm, table_hbm, out_hbm,
             table_v, idx_v, x2_v, out_v, sem_in, sem_out, *, R):
    nch = R // _C
    wid = lax.axis_index("s") * 2 + lax.axis_index("c")
    base = wid * R

    pltpu.sync_copy(table_hbm, table_v)

    def in_copy(g, b):
        row = (base + g * _C) // _C
        c_idx = pltpu.make_async_copy(x1_hbm.at[row], idx_v.at[b], sem_in.at[b])
        c_x2 = pltpu.make_async_copy(
            x2_hbm.at[pl.ds(base + g * _C, _C)], x2_v.at[b], sem_in.at[b])
        return c_idx, c_x2

    def out_copy(g, b):
        return pltpu.make_async_copy(
            out_v.at[b], out_hbm.at[pl.ds(base + g * _C, _C)], sem_out.at[b])

    for b in range(_NBUF):
        c_idx, c_x2 = in_copy(b, b)
        c_idx.start()
        c_x2.start()

    col = [lax.iota(jnp.int32, 16) + 16 * j for j in range(_D // 16)]

    def chunk_body(g2, carry):
        for b in range(_NBUF):
            g = g2 * _NBUF + b
            c_idx, c_x2 = in_copy(g, b)
            c_idx.wait()
            c_x2.wait()

            @pl.when(g2 > 0)
            def _wait_out():
                out_copy(g, b).wait()

            def row_body(r, rcarry):
                splat_r = jnp.full((16,), r, jnp.int32)
                iv = plsc.load_gather(idx_v.at[b], [splat_r])
                for j in range(_D // 16):
                    emb = plsc.load_gather(table_v, [iv, col[j]])
                    sl = pl.ds(j * 16, 16)
                    out_v[b, r, sl] = emb * x2_v[b, r, sl]
                return rcarry

            lax.fori_loop(0, _C, row_body, 0, unroll=2)

            out_copy(g, b).start()

            @pl.when(g + _NBUF < nch)
            def _start_in():
                c2_idx, c2_x2 = in_copy(g + _NBUF, b)
                c2_idx.start()
                c2_x2.start()
        return carry

    lax.fori_loop(0, nch // _NBUF, chunk_body, 0)

    for b in range(_NBUF):
        out_copy(nch - _NBUF + b, b).wait()


def kernel(x1, x2, table):
    B, L = x1.shape
    D = x2.shape[-1]
    N = B * L
    NW = 32
    R = N // NW

    x1f = x1.reshape(N // _C, _C).astype(jnp.int32)
    x2f = x2.reshape(N, D)

    mesh = plsc.VectorSubcoreMesh(core_axis_name="c", subcore_axis_name="s")
    run = functools.partial(
        pl.kernel,
        mesh=mesh,
        compiler_params=pltpu.CompilerParams(needs_layout_passes=False),
        out_type=jax.ShapeDtypeStruct((N, D), jnp.float32),
        scratch_types=[
            pltpu.VMEM((_V, _D), jnp.float32),
            pltpu.VMEM((_NBUF, _C), jnp.int32),
            pltpu.VMEM((_NBUF, _C, _D), jnp.float32),
            pltpu.VMEM((_NBUF, _C, _D), jnp.float32),
            pltpu.SemaphoreType.DMA((_NBUF,)),
            pltpu.SemaphoreType.DMA((_NBUF,)),
        ],
    )(functools.partial(_sc_body, R=R))
    out = run(x1f, x2f, table)
    return out.reshape(B, L, D)


# speedup vs baseline: 3.6780x; 3.6780x over previous
import functools

import jax
import jax.numpy as jnp
from jax import lax
from jax.experimental import pallas as pl
from jax.experimental.pallas import tpu as pltpu
from jax.experimental.pallas import tpu_sc as plsc

_C = 128
_NBUF = 2
_D = 128
_V = 64


def _sc_body(x1_hbm, x2_hbm, table_hbm, out_hbm,
             table_v, idx_v, x2_v, out_v, sem_in, sem_out, *, R):
    nch = R // _C
    wid = lax.axis_index("s") * 2 + lax.axis_index("c")
    base = wid * R

    pltpu.sync_copy(table_hbm, table_v)

    def in_copy(g, b):
        row = (base + g * _C) // _C
        c_idx = pltpu.make_async_copy(x1_hbm.at[row], idx_v.at[b], sem_in.at[b])
        c_x2 = pltpu.make_async_copy(
            x2_hbm.at[pl.ds(base + g * _C, _C)], x2_v.at[b], sem_in.at[b])
        return c_idx, c_x2

    def out_copy(g, b):
        return pltpu.make_async_copy(
            out_v.at[b], out_hbm.at[pl.ds(base + g * _C, _C)], sem_out.at[b])

    for b in range(_NBUF):
        c_idx, c_x2 = in_copy(b, b)
        c_idx.start()
        c_x2.start()

    col = [lax.iota(jnp.int32, 16) + 16 * j for j in range(_D // 16)]

    def chunk_body(g2, carry):
        for b in range(_NBUF):
            g = g2 * _NBUF + b
            c_idx, c_x2 = in_copy(g, b)
            c_idx.wait()
            c_x2.wait()

            @pl.when(g2 > 0)
            def _wait_out():
                out_copy(g, b).wait()

            @plsc.parallel_loop(0, _C, unroll=4)
            def _rows(r):
                splat_r = jnp.full((16,), r, jnp.int32)
                iv = plsc.load_gather(idx_v.at[b], [splat_r])
                for j in range(_D // 16):
                    emb = plsc.load_gather(table_v, [iv, col[j]])
                    sl = pl.ds(j * 16, 16)
                    out_v[b, r, sl] = emb * x2_v[b, r, sl]

            out_copy(g, b).start()

            @pl.when(g + _NBUF < nch)
            def _start_in():
                c2_idx, c2_x2 = in_copy(g + _NBUF, b)
                c2_idx.start()
                c2_x2.start()
        return carry

    lax.fori_loop(0, nch // _NBUF, chunk_body, 0)

    for b in range(_NBUF):
        out_copy(nch - _NBUF + b, b).wait()


def kernel(x1, x2, table):
    B, L = x1.shape
    D = x2.shape[-1]
    N = B * L
    NW = 32
    R = N // NW

    x1f = x1.reshape(N // _C, _C).astype(jnp.int32)
    x2f = x2.reshape(N, D)

    mesh = plsc.VectorSubcoreMesh(core_axis_name="c", subcore_axis_name="s")
    run = functools.partial(
        pl.kernel,
        mesh=mesh,
        compiler_params=pltpu.CompilerParams(needs_layout_passes=False),
        out_type=jax.ShapeDtypeStruct((N, D), jnp.float32),
        scratch_types=[
            pltpu.VMEM((_V, _D), jnp.float32),
            pltpu.VMEM((_NBUF, _C), jnp.int32),
            pltpu.VMEM((_NBUF, _C, _D), jnp.float32),
            pltpu.VMEM((_NBUF, _C, _D), jnp.float32),
            pltpu.SemaphoreType.DMA((_NBUF,)),
            pltpu.SemaphoreType.DMA((_NBUF,)),
        ],
    )(functools.partial(_sc_body, R=R))
    out = run(x1f, x2f, table)
    return out.reshape(B, L, D)


# speedup vs baseline: 3.7020x; 1.0065x over previous
import functools

import jax
import jax.numpy as jnp
from jax import lax
from jax.experimental import pallas as pl
from jax.experimental.pallas import tpu as pltpu
from jax.experimental.pallas import tpu_sc as plsc

_C = 128
_NBUF = 2
_D = 128
_V = 64


def _sc_body(x1_hbm, x2_hbm, table_hbm, out_hbm,
             table_v, idx_v, x2_v, out_v, sem_in, sem_out, *, R):
    nch = R // _C
    wid = lax.axis_index("s") * 2 + lax.axis_index("c")
    base = wid * R

    pltpu.sync_copy(table_hbm, table_v)

    def in_copy(g, b):
        row = (base + g * _C) // _C
        c_idx = pltpu.make_async_copy(x1_hbm.at[row], idx_v.at[b], sem_in.at[b])
        c_x2 = pltpu.make_async_copy(
            x2_hbm.at[pl.ds(base + g * _C, _C)], x2_v.at[b], sem_in.at[b])
        return c_idx, c_x2

    def out_copy(g, b):
        return pltpu.make_async_copy(
            out_v.at[b], out_hbm.at[pl.ds(base + g * _C, _C)], sem_out.at[b])

    for b in range(_NBUF):
        c_idx, c_x2 = in_copy(b, b)
        c_idx.start()
        c_x2.start()

    col = [lax.iota(jnp.int32, 16) + 16 * j for j in range(_D // 16)]

    def chunk_body(g2, carry):
        for b in range(_NBUF):
            g = g2 * _NBUF + b
            c_idx, c_x2 = in_copy(g, b)
            c_idx.wait()
            c_x2.wait()

            @pl.when(g2 > 0)
            def _wait_out():
                out_copy(g, b).wait()

            @plsc.parallel_loop(0, _C, unroll=8)
            def _rows(r):
                splat_r = jnp.full((16,), r, jnp.int32)
                iv = plsc.load_gather(idx_v.at[b], [splat_r])
                for j in range(_D // 16):
                    emb = plsc.load_gather(table_v, [iv, col[j]])
                    sl = pl.ds(j * 16, 16)
                    out_v[b, r, sl] = emb * x2_v[b, r, sl]

            out_copy(g, b).start()

            @pl.when(g + _NBUF < nch)
            def _start_in():
                c2_idx, c2_x2 = in_copy(g + _NBUF, b)
                c2_idx.start()
                c2_x2.start()
        return carry

    lax.fori_loop(0, nch // _NBUF, chunk_body, 0)

    for b in range(_NBUF):
        out_copy(nch - _NBUF + b, b).wait()


def kernel(x1, x2, table):
    B, L = x1.shape
    D = x2.shape[-1]
    N = B * L
    NW = 32
    R = N // NW

    x1f = x1.reshape(N // _C, _C).astype(jnp.int32)
    x2f = x2.reshape(N, D)

    mesh = plsc.VectorSubcoreMesh(core_axis_name="c", subcore_axis_name="s")
    run = functools.partial(
        pl.kernel,
        mesh=mesh,
        compiler_params=pltpu.CompilerParams(needs_layout_passes=False),
        out_type=jax.ShapeDtypeStruct((N, D), jnp.float32),
        scratch_types=[
            pltpu.VMEM((_V, _D), jnp.float32),
            pltpu.VMEM((_NBUF, _C), jnp.int32),
            pltpu.VMEM((_NBUF, _C, _D), jnp.float32),
            pltpu.VMEM((_NBUF, _C, _D), jnp.float32),
            pltpu.SemaphoreType.DMA((_NBUF,)),
            pltpu.SemaphoreType.DMA((_NBUF,)),
        ],
    )(functools.partial(_sc_body, R=R))
    out = run(x1f, x2f, table)
    return out.reshape(B, L, D)
